# baseline (speedup 1.0000x reference)
import functools
import jax
import jax.numpy as jnp
from jax import lax
from jax.experimental import pallas as pl
from jax.experimental.pallas import tpu as pltpu, tpu_sc as plsc

N_CLASS = 21
N_ROI = 1000
NP = 1024
CAP = 1040
IMG_H, IMG_W = 600.0, 800.0
SCORE_LOW = 0.05
NMS_THRESH = 0.3
L = 16

_mesh = plsc.VectorSubcoreMesh(core_axis_name="c", subcore_axis_name="s")


@functools.partial(
    pl.kernel,
    mesh=_mesh,
    compiler_params=pltpu.CompilerParams(needs_layout_passes=False),
    out_type=[
        jax.ShapeDtypeStruct((N_CLASS - 1, 4, NP), jnp.float32),
        jax.ShapeDtypeStruct((N_CLASS - 1, NP), jnp.float32),
    ],
    scratch_types=[
        pltpu.VMEM((4, NP), jnp.float32),
        pltpu.VMEM((4, NP), jnp.float32),
        pltpu.VMEM((N_CLASS, NP), jnp.float32),
        pltpu.VMEM((4, NP), jnp.float32),
        pltpu.VMEM((CAP,), jnp.float32),
        pltpu.VMEM((CAP,), jnp.float32),
        pltpu.VMEM((CAP,), jnp.float32),
        pltpu.VMEM((CAP,), jnp.float32),
        pltpu.VMEM((CAP,), jnp.float32),
        pltpu.VMEM((CAP,), jnp.float32),
        pltpu.VMEM((CAP,), jnp.int32),
        pltpu.VMEM((NP,), jnp.float32),
    ],
)
def _sc_nms(rois_hbm, loc_hbm, sc_hbm, boxes_out, scores_out,
            rois_v, loc_v, sc_v, box_v, y1c, x1c, y2c, x2c, areac, scc,
            idxc, out_s):
    core = lax.axis_index("c")
    sub = lax.axis_index("s")
    cls = core * 10 + sub

    @pl.when(sub < 10)
    def _():
        pltpu.sync_copy(rois_hbm, rois_v)
        pltpu.sync_copy(loc_hbm.at[cls], loc_v)
        pltpu.sync_copy(sc_hbm, sc_v)

        lane = lax.broadcasted_iota(jnp.int32, (L,), 0)

        def chunk_body(k, cnt):
            sl = pl.ds(k * L, L)
            vals = [sc_v[cc, sl] for cc in range(N_CLASS)]
            m = vals[0]
            for cc in range(1, N_CLASS):
                m = jnp.maximum(m, vals[cc])
            denom = jnp.exp(vals[0] - m)
            for cc in range(1, N_CLASS):
                denom = denom + jnp.exp(vals[cc] - m)
            s = jnp.exp(sc_v[cls + 1, sl] - m) / denom
            s = jnp.where(s > SCORE_LOW, s, 0.0)
            roi_id = lane + k * L
            s = jnp.where(roi_id < N_ROI, s, 0.0)

            ry1 = rois_v[0, sl]
            rx1 = rois_v[1, sl]
            ry2 = rois_v[2, sl]
            rx2 = rois_v[3, sl]
            sh = ry2 - ry1
            sw = rx2 - rx1
            cy = loc_v[0, sl] * sh + (ry1 + 0.5 * sh)
            cx = loc_v[1, sl] * sw + (rx1 + 0.5 * sw)
            hh = jnp.exp(loc_v[2, sl]) * sh
            ww = jnp.exp(loc_v[3, sl]) * sw
            y1 = jnp.minimum(jnp.maximum(cy - 0.5 * hh, 0.0), IMG_H)
            x1 = jnp.minimum(jnp.maximum(cx - 0.5 * ww, 0.0), IMG_W)
            y2 = jnp.minimum(jnp.maximum(cy + 0.5 * hh, 0.0), IMG_H)
            x2 = jnp.minimum(jnp.maximum(cx + 0.5 * ww, 0.0), IMG_W)
            area = jnp.maximum(y2 - y1, 0.0) * jnp.maximum(x2 - x1, 0.0)

            box_v[0, sl] = y1
            box_v[1, sl] = x1
            box_v[2, sl] = y2
            box_v[3, sl] = x2
            out_s[sl] = jnp.zeros((L,), jnp.float32)

            msk = s > 0.0
            csl = pl.ds(cnt, L)
            plsc.store_compressed(y1c.at[csl], y1, mask=msk)
            plsc.store_compressed(x1c.at[csl], x1, mask=msk)
            plsc.store_compressed(y2c.at[csl], y2, mask=msk)
            plsc.store_compressed(x2c.at[csl], x2, mask=msk)
            plsc.store_compressed(areac.at[csl], area, mask=msk)
            plsc.store_compressed(scc.at[csl], s, mask=msk)
            plsc.store_compressed(idxc.at[csl], roi_id, mask=msk)
            npop = plsc.all_reduce_population_count(msk)
            return cnt + npop[0]

        cnt = lax.fori_loop(0, NP // L, chunk_body, jnp.int32(0))
        scc[pl.ds(cnt, L)] = jnp.zeros((L,), jnp.float32)

        nch = lax.shift_right_logical(cnt + (L - 1), 4)

        def ichunk_body(t, _):
            isl = pl.ds(t * L, L)
            iy1 = y1c[isl]
            ix1 = x1c[isl]
            iy2 = y2c[isl]
            ix2 = x2c[isl]
            ia = areac[isl]
            si = scc[isl]
            ipos = lane + t * L

            def jc_body(jc, supp):
                for k in range(L):
                    jv = jnp.full((L,), jc * L + k, jnp.int32)
                    jy1 = plsc.load_gather(y1c, [jv])
                    jx1 = plsc.load_gather(x1c, [jv])
                    jy2 = plsc.load_gather(y2c, [jv])
                    jx2 = plsc.load_gather(x2c, [jv])
                    ja = plsc.load_gather(areac, [jv])
                    sj = plsc.load_gather(scc, [jv])
                    yy1 = jnp.maximum(iy1, jy1)
                    xx1 = jnp.maximum(ix1, jx1)
                    yy2 = jnp.minimum(iy2, jy2)
                    xx2 = jnp.minimum(ix2, jx2)
                    inter = jnp.maximum(yy2 - yy1, 0.0) * jnp.maximum(xx2 - xx1, 0.0)
                    union = ia + ja - inter
                    iou = inter / jnp.maximum(union, 1e-8)
                    higher = (sj > si) | ((sj == si) & (jv < ipos))
                    supp = supp | (higher & (iou > NMS_THRESH))
                return supp

            supp = lax.fori_loop(0, nch, jc_body, jnp.zeros((L,), jnp.bool_))
            outv = jnp.where(supp, 0.0, si)
            kmask = ipos < cnt
            plsc.store_scatter(out_s, [idxc[isl]], outv, mask=kmask)
            return 0

        lax.fori_loop(0, nch, ichunk_body, 0)

        pltpu.sync_copy(box_v, boxes_out.at[cls])
        pltpu.sync_copy(out_s, scores_out.at[cls])


def kernel(rois, roi_cls_loc, roi_score):
    rois = rois.astype(jnp.float32)
    loc = roi_cls_loc.astype(jnp.float32).reshape(N_ROI, N_CLASS, 4)[:, 1:, :]
    sc = roi_score.astype(jnp.float32)

    rois_T = jnp.zeros((4, NP), jnp.float32).at[:, :N_ROI].set(rois.T)
    loc_T = jnp.zeros((N_CLASS - 1, 4, NP), jnp.float32).at[:, :, :N_ROI].set(
        loc.transpose(1, 2, 0))
    sc_T = jnp.zeros((N_CLASS, NP), jnp.float32).at[:, :N_ROI].set(sc.T)

    boxes, scores = _sc_nms(rois_T, loc_T, sc_T)
    return boxes.transpose(0, 2, 1)[:, :N_ROI, :], scores[:, :N_ROI]


# speedup vs baseline: 2.2219x; 2.2219x over previous
import functools
import jax
import jax.numpy as jnp
from jax import lax
from jax.experimental import pallas as pl
from jax.experimental.pallas import tpu as pltpu, tpu_sc as plsc

N_CLASS = 21
N_ROI = 1000
NP = 1024
CAP = 1040
IMG_H, IMG_W = 600.0, 800.0
SCORE_LOW = 0.05
NMS_THRESH = 0.3
L = 16

_mesh = plsc.VectorSubcoreMesh(core_axis_name="c", subcore_axis_name="s")


@functools.partial(
    pl.kernel,
    mesh=_mesh,
    compiler_params=pltpu.CompilerParams(needs_layout_passes=False),
    out_type=[
        jax.ShapeDtypeStruct((N_CLASS - 1, 4, NP), jnp.float32),
        jax.ShapeDtypeStruct((N_CLASS - 1, NP), jnp.float32),
    ],
    scratch_types=[
        pltpu.VMEM((4, NP), jnp.float32),
        pltpu.VMEM((4, NP), jnp.float32),
        pltpu.VMEM((N_CLASS, NP), jnp.float32),
        pltpu.VMEM((4, NP), jnp.float32),
        pltpu.VMEM((CAP,), jnp.float32),
        pltpu.VMEM((CAP,), jnp.float32),
        pltpu.VMEM((CAP,), jnp.float32),
        pltpu.VMEM((CAP,), jnp.float32),
        pltpu.VMEM((CAP,), jnp.float32),
        pltpu.VMEM((CAP,), jnp.float32),
        pltpu.VMEM((CAP,), jnp.int32),
        pltpu.VMEM((NP,), jnp.float32),
    ],
)
def _sc_nms(rois_hbm, loc_hbm, sc_hbm, boxes_out, scores_out,
            rois_v, loc_v, sc_v, box_v, y1c, x1c, y2c, x2c, areac, scc,
            idxc, out_s):
    core = lax.axis_index("c")
    sub = lax.axis_index("s")
    cls = core * 10 + sub

    @pl.when(sub < 10)
    def _():
        pltpu.sync_copy(rois_hbm, rois_v)
        pltpu.sync_copy(loc_hbm.at[cls], loc_v)
        pltpu.sync_copy(sc_hbm, sc_v)

        lane = lax.broadcasted_iota(jnp.int32, (L,), 0)

        def chunk_body(k, cnt):
            sl = pl.ds(k * L, L)
            vals = [sc_v[cc, sl] for cc in range(N_CLASS)]
            m = vals[0]
            for cc in range(1, N_CLASS):
                m = jnp.maximum(m, vals[cc])
            denom = jnp.exp(vals[0] - m)
            for cc in range(1, N_CLASS):
                denom = denom + jnp.exp(vals[cc] - m)
            s = jnp.exp(sc_v[cls + 1, sl] - m) / denom
            s = jnp.where(s > SCORE_LOW, s, 0.0)
            roi_id = lane + k * L
            s = jnp.where(roi_id < N_ROI, s, 0.0)

            ry1 = rois_v[0, sl]
            rx1 = rois_v[1, sl]
            ry2 = rois_v[2, sl]
            rx2 = rois_v[3, sl]
            sh = ry2 - ry1
            sw = rx2 - rx1
            cy = loc_v[0, sl] * sh + (ry1 + 0.5 * sh)
            cx = loc_v[1, sl] * sw + (rx1 + 0.5 * sw)
            hh = jnp.exp(loc_v[2, sl]) * sh
            ww = jnp.exp(loc_v[3, sl]) * sw
            y1 = jnp.minimum(jnp.maximum(cy - 0.5 * hh, 0.0), IMG_H)
            x1 = jnp.minimum(jnp.maximum(cx - 0.5 * ww, 0.0), IMG_W)
            y2 = jnp.minimum(jnp.maximum(cy + 0.5 * hh, 0.0), IMG_H)
            x2 = jnp.minimum(jnp.maximum(cx + 0.5 * ww, 0.0), IMG_W)
            area = jnp.maximum(y2 - y1, 0.0) * jnp.maximum(x2 - x1, 0.0)

            box_v[0, sl] = y1
            box_v[1, sl] = x1
            box_v[2, sl] = y2
            box_v[3, sl] = x2
            out_s[sl] = jnp.zeros((L,), jnp.float32)

            msk = s > 0.0
            csl = pl.ds(cnt, L)
            plsc.store_compressed(y1c.at[csl], y1, mask=msk)
            plsc.store_compressed(x1c.at[csl], x1, mask=msk)
            plsc.store_compressed(y2c.at[csl], y2, mask=msk)
            plsc.store_compressed(x2c.at[csl], x2, mask=msk)
            plsc.store_compressed(areac.at[csl], area, mask=msk)
            plsc.store_compressed(scc.at[csl], s, mask=msk)
            plsc.store_compressed(idxc.at[csl], roi_id, mask=msk)
            npop = plsc.all_reduce_population_count(msk)
            return cnt + npop[0]

        cnt = lax.fori_loop(0, NP // L, chunk_body, jnp.int32(0))
        scc[pl.ds(cnt, L)] = jnp.zeros((L,), jnp.float32)

        nch = lax.shift_right_logical(cnt + (L - 1), 4)
        nq4 = lax.shift_right_logical(cnt + 3, 2)

        def ichunk_body(t, _):
            isl = pl.ds(t * L, L)
            iy1 = y1c[isl]
            ix1 = x1c[isl]
            iy2 = y2c[isl]
            ix2 = x2c[isl]
            ia = areac[isl]
            si = scc[isl]
            ipos = lane + t * L

            def jc_body(jc, supp):
                for k in range(4):
                    jv = jnp.full((L,), jc * 4 + k, jnp.int32)
                    jy1 = plsc.load_gather(y1c, [jv])
                    jx1 = plsc.load_gather(x1c, [jv])
                    jy2 = plsc.load_gather(y2c, [jv])
                    jx2 = plsc.load_gather(x2c, [jv])
                    ja = plsc.load_gather(areac, [jv])
                    sj = plsc.load_gather(scc, [jv])
                    yy1 = jnp.maximum(iy1, jy1)
                    xx1 = jnp.maximum(ix1, jx1)
                    yy2 = jnp.minimum(iy2, jy2)
                    xx2 = jnp.minimum(ix2, jx2)
                    inter = jnp.maximum(yy2 - yy1, 0.0) * jnp.maximum(xx2 - xx1, 0.0)
                    union = ia + ja - inter
                    iou = inter / jnp.maximum(union, 1e-8)
                    higher = (sj > si) | ((sj == si) & (jv < ipos))
                    supp = supp | (higher & (iou > NMS_THRESH))
                return supp

            supp = lax.fori_loop(0, nq4, jc_body, jnp.zeros((L,), jnp.bool_))
            outv = jnp.where(supp, 0.0, si)
            kmask = ipos < cnt
            plsc.store_scatter(out_s, [idxc[isl]], outv, mask=kmask)
            return 0

        lax.fori_loop(0, nch, ichunk_body, 0)

        pltpu.sync_copy(box_v, boxes_out.at[cls])
        pltpu.sync_copy(out_s, scores_out.at[cls])


def kernel(rois, roi_cls_loc, roi_score):
    rois = rois.astype(jnp.float32)
    loc = roi_cls_loc.astype(jnp.float32).reshape(N_ROI, N_CLASS, 4)[:, 1:, :]
    sc = roi_score.astype(jnp.float32)

    rois_T = jnp.zeros((4, NP), jnp.float32).at[:, :N_ROI].set(rois.T)
    loc_T = jnp.zeros((N_CLASS - 1, 4, NP), jnp.float32).at[:, :, :N_ROI].set(
        loc.transpose(1, 2, 0))
    sc_T = jnp.zeros((N_CLASS, NP), jnp.float32).at[:, :N_ROI].set(sc.T)

    boxes, scores = _sc_nms(rois_T, loc_T, sc_T)
    return boxes.transpose(0, 2, 1)[:, :N_ROI, :], scores[:, :N_ROI]
